# baseline (speedup 1.0000x reference)
import functools

import jax
import jax.numpy as jnp
from jax import lax
from jax.experimental import pallas as pl
from jax.experimental.pallas import tpu as pltpu
from jax.experimental.pallas import tpu_sc as plsc

N_NODES = 10000
N_EDGES = 320000
D_EMB = 128
D_FEAT = 6

NUM_CORES = 2
NUM_SUBCORES = 16
NUM_WORKERS = NUM_CORES * NUM_SUBCORES
EDGES_PER_WORKER = N_EDGES // NUM_WORKERS
CHUNK = 80
NUM_CHUNKS = EDGES_PER_WORKER // CHUNK
GROUPS = CHUNK // 16
NBUF = 5


def _scale_body(e_ref, w_ref, o_ref):
    o_ref[...] = e_ref[...] * w_ref[...]


def _scale_table(embedding, w128):
    return pl.pallas_call(
        _scale_body,
        out_shape=jax.ShapeDtypeStruct((N_NODES, D_EMB), jnp.float32),
    )(embedding, w128)


_mesh = plsc.VectorSubcoreMesh(core_axis_name="c", subcore_axis_name="s")


@functools.partial(
    pl.kernel,
    mesh=_mesh,
    out_type=jax.ShapeDtypeStruct((N_EDGES,), jnp.float32),
    compiler_params=pltpu.CompilerParams(needs_layout_passes=False),
    scratch_types=[
        pltpu.VMEM((EDGES_PER_WORKER,), jnp.int32),
        pltpu.VMEM((EDGES_PER_WORKER,), jnp.int32),
        pltpu.VMEM((128,), jnp.float32),
        pltpu.VMEM((NBUF * CHUNK * D_FEAT,), jnp.float32),
        pltpu.VMEM((NBUF, 2 * CHUNK, D_EMB), jnp.float32),
        pltpu.VMEM((NBUF, CHUNK), jnp.float32),
        pltpu.VMEM((16,), jnp.float32),
        pltpu.SemaphoreType.DMA((NBUF,)),
        pltpu.SemaphoreType.DMA((NBUF,)),
    ],
)
def _edge_kernel(es_hbm, e_hbm, src_hbm, dst_hbm, feats_hbm, wb_hbm, out_hbm,
                 sidx_v, didx_v, wb_v, f_v, rows_v, ob_v, tmp_v, sem_g, sem_o):
    wid = lax.axis_index("s") * NUM_CORES + lax.axis_index("c")
    ebase = wid * EDGES_PER_WORKER
    pltpu.sync_copy(wb_hbm, wb_v)
    pltpu.sync_copy(src_hbm.at[pl.ds(ebase, EDGES_PER_WORKER)], sidx_v)
    pltpu.sync_copy(dst_hbm.at[pl.ds(ebase, EDGES_PER_WORKER)], didx_v)
    lanes = lax.iota(jnp.int32, 16)

    def feat_copy(i, s):
        return pltpu.make_async_copy(
            feats_hbm.at[pl.ds((ebase + i * CHUNK) * D_FEAT, CHUNK * D_FEAT)],
            f_v.at[pl.ds(s * CHUNK * D_FEAT, CHUNK * D_FEAT)],
            sem_g.at[s])

    def src_gather(i, s):
        return pltpu.make_async_copy(
            es_hbm.at[sidx_v.at[pl.ds(i * CHUNK, CHUNK)]],
            rows_v.at[s].at[pl.ds(0, CHUNK)], sem_g.at[s])

    def dst_gather(i, s):
        return pltpu.make_async_copy(
            e_hbm.at[didx_v.at[pl.ds(i * CHUNK, CHUNK)]],
            rows_v.at[s].at[pl.ds(CHUNK, CHUNK)], sem_g.at[s])

    def out_copy(i, s):
        return pltpu.make_async_copy(
            ob_v.at[s], out_hbm.at[pl.ds(ebase + i * CHUNK, CHUNK)],
            sem_o.at[s])

    def issue(i, s):
        feat_copy(i, s).start()
        src_gather(i, s).start()
        dst_gather(i, s).start()

    def wait_in(i, s):
        feat_copy(i, s).wait()
        src_gather(i, s).wait()
        dst_gather(i, s).wait()

    def compute(i, s):
        rows2d = rows_v.at[s]
        last_lane = lanes == 15

        def group_body(g, gcarry):
            gb = g * 16
            for e in range(16):
                srow = rows2d.at[gb + e]
                drow = rows2d.at[gb + CHUNK + e]
                prods = [srow[pl.ds(u * 16, 16)] * drow[pl.ds(u * 16, 16)]
                         for u in range(8)]
                p01, p23 = prods[0] + prods[1], prods[2] + prods[3]
                p45, p67 = prods[4] + prods[5], prods[6] + prods[7]
                partial = (p01 + p23) + (p45 + p67)
                csum = plsc.cumsum(partial)
                plsc.store_scatter(tmp_v, [jnp.full((16,), e, jnp.int32)],
                                   csum, mask=last_lane)
            facc = wb_v[pl.ds(96, 16)]
            fbase = s * CHUNK * D_FEAT + (gb + lanes) * D_FEAT
            for j in range(D_FEAT):
                fv = plsc.load_gather(f_v, [fbase + j])
                facc = facc + fv * wb_v[pl.ds(j * 16, 16)]
            acc = tmp_v[...] + facc
            ob_v.at[s][pl.ds(gb, 16)] = 1.0 / (1.0 + jnp.exp(-acc))
            return gcarry

        lax.fori_loop(0, GROUPS, group_body, 0)

    issue(0, 0)
    issue(1, 1)

    def j_body(j, carry):
        for s in range(NBUF):
            i = j * NBUF + s

            @pl.when(j >= 1)
            def _():
                out_copy(i - NBUF, s).wait()

            wait_in(i, s)
            s2 = (s + 2) % NBUF
            if s < NBUF - 2:
                issue(i + 2, s2)
            else:
                @pl.when(j <= NUM_CHUNKS // NBUF - 2)
                def _():
                    issue(i + 2, s2)
            compute(i, s)
            out_copy(i, s).start()
        return carry

    lax.fori_loop(0, NUM_CHUNKS // NBUF, j_body, 0)
    for s in range(NBUF):
        out_copy(NUM_CHUNKS - NBUF + s, s).wait()


def kernel(embedding, src_id, dst_id, edge_feats, W, b):
    w128 = W[:D_EMB, 0].reshape(1, D_EMB)
    es = _scale_table(embedding, w128)
    wb = jnp.concatenate(
        [W[D_EMB:, 0], b, jnp.zeros((1,), jnp.float32)])
    wb16 = jnp.broadcast_to(wb.reshape(8, 1), (8, 16)).reshape(-1)
    out = _edge_kernel(
        es, embedding,
        src_id.astype(jnp.int32), dst_id.astype(jnp.int32),
        edge_feats.reshape(-1), wb16)
    return out.reshape(N_EDGES, 1)


# speedup vs baseline: 1.4618x; 1.4618x over previous
import functools

import jax
import jax.numpy as jnp
from jax import lax
from jax.experimental import pallas as pl
from jax.experimental.pallas import tpu as pltpu
from jax.experimental.pallas import tpu_sc as plsc

N_NODES = 10000
N_EDGES = 320000
D_EMB = 128
D_FEAT = 6

NUM_CORES = 2
NUM_SUBCORES = 16
NUM_WORKERS = NUM_CORES * NUM_SUBCORES
EDGES_PER_WORKER = N_EDGES // NUM_WORKERS
CHUNK = 80
NUM_CHUNKS = EDGES_PER_WORKER // CHUNK
GROUPS = CHUNK // 16
NBUF = 5


def _scale_body(e_ref, w_ref, o_ref):
    o_ref[...] = e_ref[...] * w_ref[...]


def _scale_table(embedding, w128):
    return pl.pallas_call(
        _scale_body,
        out_shape=jax.ShapeDtypeStruct((N_NODES, D_EMB), jnp.float32),
    )(embedding, w128)


def _featdot_body(f_ref, sw_ref, b_ref, o_ref):
    o_ref[...] = jax.lax.dot(f_ref[...], sw_ref[...],
                             precision=jax.lax.Precision.HIGHEST) + b_ref[...]


def _featdot(feats_v2, sw, b16):
    n_rows = N_EDGES // 16
    blk = n_rows // 10
    return pl.pallas_call(
        _featdot_body,
        grid=(10,),
        in_specs=[
            pl.BlockSpec((blk, 96), lambda i: (i, 0)),
            pl.BlockSpec((96, 16), lambda i: (0, 0)),
            pl.BlockSpec((1, 16), lambda i: (0, 0)),
        ],
        out_specs=pl.BlockSpec((blk, 16), lambda i: (i, 0)),
        out_shape=jax.ShapeDtypeStruct((n_rows, 16), jnp.float32),
    )(feats_v2, sw, b16)


def _combine_body(p_ref, fc_ref, o_ref):
    o_ref[...] = jax.nn.sigmoid(p_ref[...] + fc_ref[...])


def _combine(partial, fc):
    n_rows = N_EDGES // 128
    return pl.pallas_call(
        _combine_body,
        out_shape=jax.ShapeDtypeStruct((n_rows, 128), jnp.float32),
    )(partial, fc)


_mesh = plsc.VectorSubcoreMesh(core_axis_name="c", subcore_axis_name="s")


@functools.partial(
    pl.kernel,
    mesh=_mesh,
    out_type=jax.ShapeDtypeStruct((N_EDGES,), jnp.float32),
    compiler_params=pltpu.CompilerParams(needs_layout_passes=False),
    scratch_types=[
        pltpu.VMEM((EDGES_PER_WORKER,), jnp.int32),
        pltpu.VMEM((EDGES_PER_WORKER,), jnp.int32),
        pltpu.VMEM((NBUF, 2 * CHUNK, D_EMB), jnp.float32),
        pltpu.VMEM((NBUF, CHUNK), jnp.float32),
        pltpu.VMEM((16,), jnp.float32),
        pltpu.SemaphoreType.DMA((NBUF,)),
        pltpu.SemaphoreType.DMA((NBUF,)),
    ],
)
def _edge_kernel(es_hbm, e_hbm, src_hbm, dst_hbm, out_hbm,
                 sidx_v, didx_v, rows_v, ob_v, tmp_v, sem_g, sem_o):
    wid = lax.axis_index("s") * NUM_CORES + lax.axis_index("c")
    ebase = wid * EDGES_PER_WORKER
    pltpu.sync_copy(src_hbm.at[pl.ds(ebase, EDGES_PER_WORKER)], sidx_v)
    pltpu.sync_copy(dst_hbm.at[pl.ds(ebase, EDGES_PER_WORKER)], didx_v)
    lanes = lax.iota(jnp.int32, 16)

    def src_gather(i, s):
        return pltpu.make_async_copy(
            es_hbm.at[sidx_v.at[pl.ds(i * CHUNK, CHUNK)]],
            rows_v.at[s].at[pl.ds(0, CHUNK)], sem_g.at[s])

    def dst_gather(i, s):
        return pltpu.make_async_copy(
            e_hbm.at[didx_v.at[pl.ds(i * CHUNK, CHUNK)]],
            rows_v.at[s].at[pl.ds(CHUNK, CHUNK)], sem_g.at[s])

    def out_copy(i, s):
        return pltpu.make_async_copy(
            ob_v.at[s], out_hbm.at[pl.ds(ebase + i * CHUNK, CHUNK)],
            sem_o.at[s])

    def issue(i, s):
        src_gather(i, s).start()
        dst_gather(i, s).start()

    def wait_in(i, s):
        src_gather(i, s).wait()
        dst_gather(i, s).wait()

    def compute(i, s):
        rows2d = rows_v.at[s]
        last_lane = lanes == 15

        def group_body(g, gcarry):
            gb = g * 16
            for e in range(16):
                srow = rows2d.at[gb + e]
                drow = rows2d.at[gb + CHUNK + e]
                prods = [srow[pl.ds(u * 16, 16)] * drow[pl.ds(u * 16, 16)]
                         for u in range(8)]
                p01, p23 = prods[0] + prods[1], prods[2] + prods[3]
                p45, p67 = prods[4] + prods[5], prods[6] + prods[7]
                partial = (p01 + p23) + (p45 + p67)
                csum = plsc.cumsum(partial)
                plsc.store_scatter(tmp_v, [jnp.full((16,), e, jnp.int32)],
                                   csum, mask=last_lane)
            ob_v.at[s][pl.ds(gb, 16)] = tmp_v[...]
            return gcarry

        lax.fori_loop(0, GROUPS, group_body, 0)

    issue(0, 0)
    issue(1, 1)

    def j_body(j, carry):
        for s in range(NBUF):
            i = j * NBUF + s

            @pl.when(j >= 1)
            def _():
                out_copy(i - NBUF, s).wait()

            wait_in(i, s)
            s2 = (s + 2) % NBUF
            if s < NBUF - 2:
                issue(i + 2, s2)
            else:
                @pl.when(j <= NUM_CHUNKS // NBUF - 2)
                def _():
                    issue(i + 2, s2)
            compute(i, s)
            out_copy(i, s).start()
        return carry

    lax.fori_loop(0, NUM_CHUNKS // NBUF, j_body, 0)
    for s in range(NBUF):
        out_copy(NUM_CHUNKS - NBUF + s, s).wait()


def kernel(embedding, src_id, dst_id, edge_feats, W, b):
    w128 = W[:D_EMB, 0].reshape(1, D_EMB)
    es = _scale_table(embedding, w128)
    partial = _edge_kernel(
        es, embedding,
        src_id.astype(jnp.int32), dst_id.astype(jnp.int32))
    w6 = W[D_EMB:, 0]
    sw = jnp.kron(jnp.eye(16, dtype=jnp.float32), w6.reshape(D_FEAT, 1))
    b16 = jnp.broadcast_to(b, (1, 16))
    fc = _featdot(edge_feats.reshape(N_EDGES // 16, 16 * D_FEAT), sw, b16)
    out = _combine(partial.reshape(N_EDGES // 128, 128),
                   fc.reshape(N_EDGES // 128, 128))
    return out.reshape(N_EDGES, 1)
